# baseline (speedup 1.0000x reference)
import functools

import jax
import jax.numpy as jnp
from jax import lax
from jax.experimental import pallas as pl
from jax.experimental.pallas import tpu as pltpu
from jax.experimental.pallas import tpu_sc as plsc

_L = 128


@functools.cache
def _build(N, S, V, D):
    info = plsc.get_sparse_core_info()
    NC, NS = info.num_cores, info.num_subcores
    NW = NC * NS
    assert N % (_L * NW) == 0 and D % 8 == 0 and S % 8 == 0
    NB = N // _L
    ST = S // 8

    mesh = plsc.VectorSubcoreMesh(core_axis_name="c", subcore_axis_name="s")

    @functools.partial(
        pl.kernel,
        out_type=jax.ShapeDtypeStruct((S, D // 8, NB, 8, _L), jnp.float32),
        mesh=mesh,
        scratch_types=[
            pltpu.VMEM((ST, 1, 8, _L), jnp.int32),
            pltpu.VMEM((_L, D), jnp.float32),
            pltpu.VMEM((_L, D), jnp.float32),
            pltpu.VMEM((D // 8, 1, 8, _L + 1), jnp.float32),
            pltpu.VMEM((D // 8, 1, 8, _L + 1), jnp.float32),
            pltpu.SemaphoreType.DMA,
            pltpu.SemaphoreType.DMA,
            pltpu.SemaphoreType.DMA,
            pltpu.SemaphoreType.DMA,
        ],
        compiler_params=pltpu.CompilerParams(use_tc_tiling_on_sc=False,
                                             needs_layout_passes=False),
    )
    def gather_kernel(idx_hbm, table_hbm, out_hbm, idx_v, r0, r1, t0, t1,
                      g0, g1, w0, w1):
        wid = lax.axis_index("s") * NC + lax.axis_index("c")
        pltpu.sync_copy(idx_hbm.at[:, pl.ds(wid, 1)], idx_v)

        iot = lax.iota(jnp.int32, 16)
        zero16 = jnp.zeros((16,), jnp.int32)
        dts = [(iot + 16 * k) // 8 for k in range(D // 16)]
        drs = [lax.rem(iot + 16 * k, 8) for k in range(D // 16)]

        def fire_gather(s, rbuf, sem):
            pltpu.async_copy(table_hbm.at[idx_v.at[s // 8, 0, s % 8]],
                             rbuf, sem)

        def drain_gather(rbuf, sem):
            pltpu.make_async_copy(table_hbm.at[pl.ds(0, _L)], rbuf,
                                  sem).wait()

        def transpose(rbuf, tbuf):
            @plsc.parallel_loop(0, _L, unroll=8)
            def _(n):
                lane = jnp.full((16,), n, jnp.int32)
                for k in range(D // 16):
                    v = rbuf[n, pl.ds(k * 16, 16)]
                    plsc.store_scatter(tbuf, [dts[k], zero16, drs[k], lane],
                                       v)

        def fire_write(s, tbuf, sem):
            pltpu.async_copy(tbuf.at[:, :, :, pl.ds(0, _L)],
                             out_hbm.at[s, :, pl.ds(wid, 1)], sem)

        def drain_write(tbuf, sem):
            pltpu.make_async_copy(tbuf.at[:, :, :, pl.ds(0, _L)],
                                  out_hbm.at[0, :, pl.ds(wid, 1)],
                                  sem).wait()

        fire_gather(0, r0, g0)

        @pl.loop(0, S, step=2)
        def _(s):
            fire_gather(s + 1, r1, g1)
            drain_gather(r0, g0)

            @pl.when(s >= 2)
            def _():
                drain_write(t0, w0)

            transpose(r0, t0)
            fire_write(s, t0, w0)

            @pl.when(s + 2 < S)
            def _():
                fire_gather(s + 2, r0, g0)

            drain_gather(r1, g1)

            @pl.when(s >= 2)
            def _():
                drain_write(t1, w1)

            transpose(r1, t1)
            fire_write(s + 1, t1, w1)

        drain_write(t0, w0)
        drain_write(t1, w1)

    return gather_kernel


def kernel(idx, pe):
    N, S = idx.shape
    V, D = pe.shape
    idx_t = (idx.astype(jnp.int32)
             .reshape(N // _L, _L, S // 8, 8)
             .transpose(2, 0, 3, 1))
    out5 = _build(N, S, V, D)(idx_t, pe)
    return out5.transpose(2, 4, 0, 1, 3).reshape(N, S, D)


# speedup vs baseline: 1.1758x; 1.1758x over previous
import functools

import jax
import jax.numpy as jnp
from jax import lax
from jax.experimental import pallas as pl
from jax.experimental.pallas import tpu as pltpu
from jax.experimental.pallas import tpu_sc as plsc

_L = 128


@functools.cache
def _build(N, S, V, D):
    info = plsc.get_sparse_core_info()
    NC, NS = info.num_cores, info.num_subcores
    NW = NC * NS
    assert N % (_L * NW) == 0 and D % 8 == 0 and S % 8 == 0
    NB = N // _L
    ST = S // 8

    mesh = plsc.VectorSubcoreMesh(core_axis_name="c", subcore_axis_name="s")

    @functools.partial(
        pl.kernel,
        out_type=jax.ShapeDtypeStruct((S, D // 8, NB, 8, _L), jnp.float32),
        mesh=mesh,
        scratch_types=[
            pltpu.VMEM((ST, 1, 8, _L), jnp.int32),
            pltpu.VMEM((_L, D), jnp.float32),
            pltpu.VMEM((_L, D), jnp.float32),
            pltpu.VMEM((_L, D), jnp.float32),
            pltpu.VMEM((_L, D), jnp.float32),
            pltpu.VMEM((D // 8, 1, 8, _L + 1), jnp.float32),
            pltpu.VMEM((D // 8, 1, 8, _L + 1), jnp.float32),
            pltpu.SemaphoreType.DMA,
            pltpu.SemaphoreType.DMA,
            pltpu.SemaphoreType.DMA,
            pltpu.SemaphoreType.DMA,
            pltpu.SemaphoreType.DMA,
            pltpu.SemaphoreType.DMA,
        ],
        compiler_params=pltpu.CompilerParams(use_tc_tiling_on_sc=False,
                                             needs_layout_passes=False),
    )
    def gather_kernel(idx_hbm, table_hbm, out_hbm, idx_v, r0, r1, r2, r3,
                      t0, t1, g0, g1, g2, g3, w0, w1):
        wid = lax.axis_index("s") * NC + lax.axis_index("c")
        pltpu.sync_copy(idx_hbm.at[:, pl.ds(wid, 1)], idx_v)

        iot = lax.iota(jnp.int32, 16)
        zero16 = jnp.zeros((16,), jnp.int32)
        dts = [(iot + 16 * k) // 8 for k in range(D // 16)]
        drs = [lax.rem(iot + 16 * k, 8) for k in range(D // 16)]

        def fire_gather(s, rbuf, sem):
            pltpu.async_copy(table_hbm.at[idx_v.at[s // 8, 0, s % 8]],
                             rbuf, sem)

        def drain_gather(rbuf, sem):
            pltpu.make_async_copy(table_hbm.at[pl.ds(0, _L)], rbuf,
                                  sem).wait()

        def transpose(rbuf, tbuf):
            @plsc.parallel_loop(0, _L, unroll=8)
            def _(n):
                lane = jnp.full((16,), n, jnp.int32)
                for k in range(D // 16):
                    v = rbuf[n, pl.ds(k * 16, 16)]
                    plsc.store_scatter(tbuf, [dts[k], zero16, drs[k], lane],
                                       v)

        def fire_write(s, tbuf, sem):
            pltpu.async_copy(tbuf.at[:, :, :, pl.ds(0, _L)],
                             out_hbm.at[s, :, pl.ds(wid, 1)], sem)

        def drain_write(tbuf, sem):
            pltpu.make_async_copy(tbuf.at[:, :, :, pl.ds(0, _L)],
                                  out_hbm.at[0, :, pl.ds(wid, 1)],
                                  sem).wait()

        rs = [r0, r1, r2, r3]
        gs = [g0, g1, g2, g3]
        ts = [t0, t1]
        ws = [w0, w1]

        fire_gather(0, r0, g0)
        fire_gather(1, r1, g1)
        fire_gather(2, r2, g2)

        @pl.loop(0, S, step=4)
        def _(s):
            for j in range(4):
                sb = s + j

                @pl.when(sb + 3 < S)
                def _():
                    fire_gather(sb + 3, rs[(j + 3) % 4], gs[(j + 3) % 4])

                drain_gather(rs[j], gs[j])

                @pl.when(sb >= 2)
                def _():
                    drain_write(ts[j % 2], ws[j % 2])

                transpose(rs[j], ts[j % 2])
                fire_write(sb, ts[j % 2], ws[j % 2])

        drain_write(t0, w0)
        drain_write(t1, w1)

    return gather_kernel


def kernel(idx, pe):
    N, S = idx.shape
    V, D = pe.shape
    idx_t = (idx.astype(jnp.int32)
             .reshape(N // _L, _L, S // 8, 8)
             .transpose(2, 0, 3, 1))
    out5 = _build(N, S, V, D)(idx_t, pe)
    return out5.transpose(2, 4, 0, 1, 3).reshape(N, S, D)
